# Initial kernel scaffold; baseline (speedup 1.0000x reference)
import jax
import jax.numpy as jnp
from jax.experimental import pallas as pl


def kernel(x, norm_scale, gate_w, mlp1_w, mlp1_b, mlp2_w, mlp2_b):
    raise NotImplementedError("write your pallas kernel here")


# speedup vs baseline: 3.5036x; 3.5036x over previous
import jax
import jax.numpy as jnp
from jax.experimental import pallas as pl
from jax.experimental.pallas import tpu as pltpu

_S = 64
_H = 512
_I = 512
_E = 16
_ALPHA = 1.702
_LIMIT = 7.0
_EPS = 1e-5


def _moe_kernel(x_ref, scale_ref, gate_ref, w1_ref, b1g_ref,
                b1l_ref, w2_ref, b2_ref, out_ref, t_ref, rw_ref):
    e = pl.program_id(0)

    @pl.when(e == 0)
    def _prologue():
        x = x_ref[...]
        v = jnp.mean(x * x, axis=-1, keepdims=True)
        t = x * jax.lax.rsqrt(v + _EPS) * scale_ref[...]
        t_ref[...] = t
        g = jax.lax.dot_general(t, gate_ref[...], (((1,), (1,)), ((), ())),
                                preferred_element_type=jnp.float32)
        iota = jax.lax.broadcasted_iota(jnp.int32, (_S, _E), 1)
        v1 = jnp.max(g, axis=1, keepdims=True)
        i1 = jnp.min(jnp.where(g == v1, iota, _E), axis=1, keepdims=True)
        m1 = iota == i1
        gm = jnp.where(m1, -jnp.inf, g)
        v2 = jnp.max(gm, axis=1, keepdims=True)
        i2 = jnp.min(jnp.where(gm == v2, iota, _E), axis=1, keepdims=True)
        m2 = iota == i2
        p1 = jax.nn.sigmoid(v1 - v2)
        rw_ref[...] = jnp.where(m1, p1, 0.0) + jnp.where(m2, 1.0 - p1, 0.0)
        out_ref[...] = x

    t = t_ref[...]
    w1g = w1_ref[0, :, 0, :]
    w1l = w1_ref[0, :, 1, :]
    hg = jax.lax.dot_general(t, w1g, (((1,), (1,)), ((), ())),
                             preferred_element_type=jnp.float32) + b1g_ref[0]
    hl = jax.lax.dot_general(t, w1l, (((1,), (1,)), ((), ())),
                             preferred_element_type=jnp.float32) + b1l_ref[0]
    hg = jnp.minimum(hg, _LIMIT)
    hl = jnp.clip(hl, -_LIMIT, _LIMIT)
    act = hg * jax.nn.sigmoid(_ALPHA * hg) * (hl + 1.0)
    o = jax.lax.dot_general(act, w2_ref[0], (((1,), (1,)), ((), ())),
                            preferred_element_type=jnp.float32) + b2_ref[0]
    iota = jax.lax.broadcasted_iota(jnp.int32, (_S, _E), 1)
    w_col = jnp.sum(jnp.where(iota == e, rw_ref[...], 0.0), axis=1,
                    keepdims=True)
    out_ref[...] += o * w_col


def kernel(x, norm_scale, gate_w, mlp1_w, mlp1_b, mlp2_w, mlp2_b):
    w1v = mlp1_w.reshape(_E, _I, 2, _H)
    b1v = mlp1_b.reshape(_E, _I, 2)
    b1g = b1v[:, :, 0].reshape(_E, 1, _I)
    b1l = b1v[:, :, 1].reshape(_E, 1, _I)
    b2v = mlp2_b.reshape(_E, 1, _H)
    scale2d = norm_scale.reshape(1, _H)

    in_specs = [
            pl.BlockSpec((_S, _H), lambda e: (0, 0)),
            pl.BlockSpec((1, _H), lambda e: (0, 0)),
            pl.BlockSpec((_E, _H), lambda e: (0, 0)),
            pl.BlockSpec((1, _I, 2, _H), lambda e: (e, 0, 0, 0)),
            pl.BlockSpec((1, 1, _I), lambda e: (e, 0, 0)),
            pl.BlockSpec((1, 1, _I), lambda e: (e, 0, 0)),
            pl.BlockSpec((1, _H, _I), lambda e: (e, 0, 0)),
            pl.BlockSpec((1, 1, _H), lambda e: (e, 0, 0)),
    ]
    return pl.pallas_call(
        _moe_kernel,
        grid=(_E,),
        in_specs=in_specs,
        out_specs=pl.BlockSpec((_S, _H), lambda e: (0, 0)),
        out_shape=jax.ShapeDtypeStruct((_S, _H), jnp.float32),
        scratch_shapes=[
            pltpu.VMEM((_S, _H), jnp.float32),
            pltpu.VMEM((_S, _E), jnp.float32),
        ],
        compiler_params=pltpu.CompilerParams(
            dimension_semantics=("arbitrary",),
        ),
    )(x, scale2d, gate_w, w1v, b1g, b1l, mlp2_w, b2v)


# speedup vs baseline: 8.0252x; 2.2906x over previous
import jax
import jax.numpy as jnp
from jax.experimental import pallas as pl
from jax.experimental.pallas import tpu as pltpu

_S = 64
_H = 512
_I = 512
_E = 16
_ALPHA = 1.702
_LIMIT = 7.0
_EPS = 1e-5


def _moe_kernel(x_ref, scale_ref, gate_ref, w1_ref, b1_ref, w2_ref, b2_ref,
                out_ref, t_ref, rw_ref, p_ref):
    e = pl.program_id(0)

    @pl.when(e == 0)
    def _prologue():
        x = x_ref[...]
        v = jnp.mean(x * x, axis=-1, keepdims=True)
        t = x * jax.lax.rsqrt(v + _EPS) * scale_ref[...]
        t_ref[...] = t
        g = jax.lax.dot_general(t, gate_ref[...], (((1,), (1,)), ((), ())),
                                preferred_element_type=jnp.float32)
        iota = jax.lax.broadcasted_iota(jnp.int32, (_S, _E), 1)
        v1 = jnp.max(g, axis=1, keepdims=True)
        i1 = jnp.min(jnp.where(g == v1, iota, _E), axis=1, keepdims=True)
        m1 = iota == i1
        gm = jnp.where(m1, -jnp.inf, g)
        v2 = jnp.max(gm, axis=1, keepdims=True)
        i2 = jnp.min(jnp.where(gm == v2, iota, _E), axis=1, keepdims=True)
        m2 = iota == i2
        p1 = jax.nn.sigmoid(v1 - v2)
        rw_ref[...] = jnp.where(m1, p1, 0.0) + jnp.where(m2, 1.0 - p1, 0.0)
        r = jax.lax.broadcasted_iota(jnp.int32, (2 * _I, 2 * _I), 0)
        c = jax.lax.broadcasted_iota(jnp.int32, (2 * _I, 2 * _I), 1)
        src = jnp.where(c < _I, 2 * c, 2 * c - (2 * _I - 1))
        p_ref[...] = (r == src).astype(jnp.float32)
        out_ref[...] = x

    t = t_ref[...]
    h = jax.lax.dot_general(t, w1_ref[0], (((1,), (1,)), ((), ())),
                            preferred_element_type=jnp.float32) + b1_ref[0]
    hp = jax.lax.dot_general(h, p_ref[...], (((1,), (0,)), ((), ())),
                             preferred_element_type=jnp.float32)
    hg = hp[:, :_I]
    hl = hp[:, _I:]
    hg = jnp.minimum(hg, _LIMIT)
    hl = jnp.clip(hl, -_LIMIT, _LIMIT)
    act = hg * jax.nn.sigmoid(_ALPHA * hg) * (hl + 1.0)
    o = jax.lax.dot_general(act, w2_ref[0], (((1,), (1,)), ((), ())),
                            preferred_element_type=jnp.float32) + b2_ref[0]
    iota = jax.lax.broadcasted_iota(jnp.int32, (_S, _E), 1)
    w_col = jnp.sum(jnp.where(iota == e, rw_ref[...], 0.0), axis=1,
                    keepdims=True)
    out_ref[...] += o * w_col


def kernel(x, norm_scale, gate_w, mlp1_w, mlp1_b, mlp2_w, mlp2_b):
    b1v = mlp1_b.reshape(_E, 1, 2 * _I)
    b2v = mlp2_b.reshape(_E, 1, _H)
    scale2d = norm_scale.reshape(1, _H)

    in_specs = [
            pl.BlockSpec((_S, _H), lambda e: (0, 0)),
            pl.BlockSpec((1, _H), lambda e: (0, 0)),
            pl.BlockSpec((_E, _H), lambda e: (0, 0)),
            pl.BlockSpec((1, 2 * _I, _H), lambda e: (e, 0, 0)),
            pl.BlockSpec((1, 1, 2 * _I), lambda e: (e, 0, 0)),
            pl.BlockSpec((1, _H, _I), lambda e: (e, 0, 0)),
            pl.BlockSpec((1, 1, _H), lambda e: (e, 0, 0)),
    ]
    return pl.pallas_call(
        _moe_kernel,
        grid=(_E,),
        in_specs=in_specs,
        out_specs=pl.BlockSpec((_S, _H), lambda e: (0, 0)),
        out_shape=jax.ShapeDtypeStruct((_S, _H), jnp.float32),
        scratch_shapes=[
            pltpu.VMEM((_S, _H), jnp.float32),
            pltpu.VMEM((_S, _E), jnp.float32),
            pltpu.VMEM((2 * _I, 2 * _I), jnp.float32),
        ],
        compiler_params=pltpu.CompilerParams(
            dimension_semantics=("arbitrary",),
        ),
    )(x, scale2d, gate_w, mlp1_w, b1v, mlp2_w, b2v)


# speedup vs baseline: 8.8788x; 1.1064x over previous
import jax
import jax.numpy as jnp
from jax.experimental import pallas as pl
from jax.experimental.pallas import tpu as pltpu

_S = 64
_H = 512
_I = 512
_E = 16
_G = 2
_ALPHA = 1.702
_LIMIT = 7.0
_EPS = 1e-5


def _moe_kernel(x_ref, scale_ref, gate_ref, w1_ref, b1_ref, w2_ref, b2_ref,
                out_ref, t_ref, rw_ref, p_ref):
    e = pl.program_id(0)

    @pl.when(e == 0)
    def _prologue():
        x = x_ref[...]
        v = jnp.mean(x * x, axis=-1, keepdims=True)
        t = x * jax.lax.rsqrt(v + _EPS) * scale_ref[...]
        t_ref[...] = t
        g = jax.lax.dot_general(t, gate_ref[...], (((1,), (1,)), ((), ())),
                                preferred_element_type=jnp.float32)
        iota = jax.lax.broadcasted_iota(jnp.int32, (_S, _E), 1)
        v1 = jnp.max(g, axis=1, keepdims=True)
        i1 = jnp.min(jnp.where(g == v1, iota, _E), axis=1, keepdims=True)
        m1 = iota == i1
        gm = jnp.where(m1, -jnp.inf, g)
        v2 = jnp.max(gm, axis=1, keepdims=True)
        i2 = jnp.min(jnp.where(gm == v2, iota, _E), axis=1, keepdims=True)
        m2 = iota == i2
        p1 = jax.nn.sigmoid(v1 - v2)
        rw_ref[...] = jnp.where(m1, p1, 0.0) + jnp.where(m2, 1.0 - p1, 0.0)
        r = jax.lax.broadcasted_iota(jnp.int32, (2 * _I, 2 * _I), 0)
        c = jax.lax.broadcasted_iota(jnp.int32, (2 * _I, 2 * _I), 1)
        src = jnp.where(c < _I, 2 * c, 2 * c - (2 * _I - 1))
        p_ref[...] = (r == src).astype(jnp.float32)
        out_ref[...] = x

    t = t_ref[...]
    iota = jax.lax.broadcasted_iota(jnp.int32, (_S, _E), 1)
    rw = rw_ref[...]
    acc = out_ref[...]
    for j in range(_G):
        h = jax.lax.dot_general(t, w1_ref[j], (((1,), (1,)), ((), ())),
                                preferred_element_type=jnp.float32) + b1_ref[j]
        hp = jax.lax.dot_general(h, p_ref[...], (((1,), (0,)), ((), ())),
                                 preferred_element_type=jnp.float32)
        hg = hp[:, :_I]
        hl = hp[:, _I:]
        hg = jnp.minimum(hg, _LIMIT)
        hl = jnp.clip(hl, -_LIMIT, _LIMIT)
        act = hg * jax.nn.sigmoid(_ALPHA * hg) * (hl + 1.0)
        o = jax.lax.dot_general(act, w2_ref[j], (((1,), (1,)), ((), ())),
                                preferred_element_type=jnp.float32) + b2_ref[j]
        w_col = jnp.sum(jnp.where(iota == e * _G + j, rw, 0.0), axis=1,
                        keepdims=True)
        acc = acc + o * w_col
    out_ref[...] = acc


def kernel(x, norm_scale, gate_w, mlp1_w, mlp1_b, mlp2_w, mlp2_b):
    b1v = mlp1_b.reshape(_E, 1, 2 * _I)
    b2v = mlp2_b.reshape(_E, 1, _H)
    scale2d = norm_scale.reshape(1, _H)

    in_specs = [
            pl.BlockSpec((_S, _H), lambda e: (0, 0)),
            pl.BlockSpec((1, _H), lambda e: (0, 0)),
            pl.BlockSpec((_E, _H), lambda e: (0, 0)),
            pl.BlockSpec((_G, 2 * _I, _H), lambda e: (e, 0, 0)),
            pl.BlockSpec((_G, 1, 2 * _I), lambda e: (e, 0, 0)),
            pl.BlockSpec((_G, _H, _I), lambda e: (e, 0, 0)),
            pl.BlockSpec((_G, 1, _H), lambda e: (e, 0, 0)),
    ]
    return pl.pallas_call(
        _moe_kernel,
        grid=(_E // _G,),
        in_specs=in_specs,
        out_specs=pl.BlockSpec((_S, _H), lambda e: (0, 0)),
        out_shape=jax.ShapeDtypeStruct((_S, _H), jnp.float32),
        scratch_shapes=[
            pltpu.VMEM((_S, _H), jnp.float32),
            pltpu.VMEM((_S, _E), jnp.float32),
            pltpu.VMEM((2 * _I, 2 * _I), jnp.float32),
        ],
        compiler_params=pltpu.CompilerParams(
            dimension_semantics=("arbitrary",),
        ),
    )(x, scale2d, gate_w, mlp1_w, b1v, mlp2_w, b2v)
